# Initial kernel scaffold; baseline (speedup 1.0000x reference)
import functools

import jax
import jax.numpy as jnp
from jax import lax
from jax.experimental import pallas as pl
from jax.experimental.pallas import tpu as pltpu
from jax.experimental.pallas import tpu_sc as plsc

N_MET = 100000
N_RXN = 100000
E_SUB = 1600000
E_ALL = 3200000

N_MET_P = 100352
N_RXN_P = 100352
NC, NS = 2, 16
NW = NC * NS
CH = 128
KB = 8
SUB_ROWS_W = 392
ALL_ROWS_W = 784
E_SUB_P = NW * SUB_ROWS_W * CH
E_ALL_P = NW * ALL_ROWS_W * CH
BLK = 2048
EBLK = 32768

_mesh = plsc.VectorSubcoreMesh(core_axis_name="c", subcore_axis_name="s")


def _prep_sub_body(met_ref, sto_ref, o_ref):
    o_ref[...] = met_ref[...] + jnp.where(sto_ref[...] > 1.5, N_MET_P, 0).astype(jnp.int32)


def _prep_all_body(rxn_ref, sto_ref, o_ref):
    s = sto_ref[...]
    code = (jnp.where(jnp.abs(s) > 1.5, 2, 0) + jnp.where(s < 0.0, 1, 0)).astype(jnp.int32)
    o_ref[...] = rxn_ref[...] + code * N_RXN_P


def _stage_a_body(x_ref, W1_ref, b1_ref, W2_ref, b2_ref, o_ref):
    i = pl.program_id(0)
    xb = x_ref[...]
    conc = xb[:, 3:4]
    extc = xb[:, 4:5] * 2.0
    rows = i * BLK + lax.broadcasted_iota(jnp.int32, (BLK, 1), 0)
    mask = (rows < N_MET).astype(jnp.float32)
    W1 = W1_ref[...]
    outs = []
    for s in (1.0, 2.0):
        h = jnp.tanh(conc * W1[0:1, :] + s * W1[1:2, :] + b1_ref[...][None, :])
        msg = h @ W2_ref[...] + b2_ref[...][None, :]
        outs.append(jnp.concatenate([msg * mask, extc * mask, mask], axis=1))
    o_ref[...] = jnp.stack(outs, axis=0)


def _stage_c_body(a_ref, R1_ref, rb1_ref, R2_ref, rb2_ref, lk_ref, o_ref):
    a = a_ref[0] + a_ref[1]
    hh = jnp.tanh(a[:, 0:4] @ R1_ref[...] + rb1_ref[...][None, :])
    base = hh @ R2_ref[...] + rb2_ref[...][None, :]
    ext_mean = a[:, 4:5] / jnp.maximum(a[:, 5:6], 1.0)
    k = jnp.power(10.0, lk_ref[...][:, None])
    v = (k * ext_mean * base)[:, 0]
    o_ref[...] = jnp.stack([v, -v, 2.0 * v, -2.0 * v], axis=0)


def _stage_e_body(p_ref, conc_ref, o_ref):
    o_ref[...] = p_ref[0] + p_ref[1] - 0.1 * (conc_ref[...] - 1.0)


def _sc_b_body(T6_hbm, gidx_hbm, rxn_hbm, z_hbm, out_hbm, T6_sp, A6_sp, gbuf, rbuf, rows_buf):
    cid = lax.axis_index("c")
    sid = lax.axis_index("s")
    wid = sid * NC + cid
    t_stripe = 2 * N_MET_P // NS
    a_stripe = N_RXN_P // NS
    pltpu.sync_copy(T6_hbm.at[pl.ds(sid * t_stripe, t_stripe)],
                    T6_sp.at[pl.ds(sid * t_stripe, t_stripe)])
    pltpu.sync_copy(z_hbm, A6_sp.at[pl.ds(sid * a_stripe, a_stripe)])
    plsc.subcore_barrier()

    @pl.loop(0, SUB_ROWS_W // KB)
    def _(blk):
        row0 = wid * SUB_ROWS_W + blk * KB
        pltpu.sync_copy(gidx_hbm.at[pl.ds(row0, KB)], gbuf)
        pltpu.sync_copy(rxn_hbm.at[pl.ds(row0, KB)], rbuf)
        for j in range(KB):
            pltpu.sync_copy(T6_sp.at[gbuf.at[j]], rows_buf)
            pltpu.sync_copy(rows_buf, A6_sp.at[rbuf.at[j]], add=True)

    plsc.subcore_barrier()
    pltpu.sync_copy(A6_sp.at[pl.ds(sid * a_stripe, a_stripe)],
                    out_hbm.at[cid, pl.ds(sid * a_stripe, a_stripe)])


def _sc_d_body(v4_hbm, gidx_hbm, met_hbm, z_hbm, out_hbm, v4_sp, dx_sp, gbuf, mbuf, vals_buf):
    cid = lax.axis_index("c")
    sid = lax.axis_index("s")
    wid = sid * NC + cid
    v_stripe = 4 * N_RXN_P // NS
    d_stripe = N_MET_P // NS
    pltpu.sync_copy(v4_hbm.at[pl.ds(sid * v_stripe, v_stripe)],
                    v4_sp.at[pl.ds(sid * v_stripe, v_stripe)])
    pltpu.sync_copy(z_hbm, dx_sp.at[pl.ds(sid * d_stripe, d_stripe)])
    plsc.subcore_barrier()

    @pl.loop(0, ALL_ROWS_W // KB)
    def _(blk):
        row0 = wid * ALL_ROWS_W + blk * KB
        pltpu.sync_copy(gidx_hbm.at[pl.ds(row0, KB)], gbuf)
        pltpu.sync_copy(met_hbm.at[pl.ds(row0, KB)], mbuf)
        for j in range(KB):
            pltpu.sync_copy(v4_sp.at[gbuf.at[j]], vals_buf)
            pltpu.sync_copy(vals_buf, dx_sp.at[mbuf.at[j]], add=True)

    plsc.subcore_barrier()
    pltpu.sync_copy(dx_sp.at[pl.ds(sid * d_stripe, d_stripe)],
                    out_hbm.at[cid, pl.ds(sid * d_stripe, d_stripe)])


def kernel(x, met_sub, rxn_sub, sto_sub, met_all, rxn_all, sto_all,
           W1, b1, W2, b2, R1, rb1, R2, rb2, log_k):
    f32 = jnp.float32
    i32 = jnp.int32

    x_p = jnp.pad(x.astype(f32), ((0, N_MET_P - N_MET), (0, 0)))
    met_sub_p = jnp.pad(met_sub.astype(i32), (0, E_SUB_P - E_SUB), constant_values=N_MET)
    sto_sub_p = jnp.pad(sto_sub.astype(f32), (0, E_SUB_P - E_SUB), constant_values=1.0)
    rxn_sub_p = jnp.pad(rxn_sub.astype(i32), (0, E_SUB_P - E_SUB), constant_values=N_RXN)
    met_all_p = jnp.pad(met_all.astype(i32), (0, E_ALL_P - E_ALL), constant_values=N_MET)
    rxn_all_p = jnp.pad(rxn_all.astype(i32), (0, E_ALL_P - E_ALL), constant_values=N_RXN)
    sto_all_p = jnp.pad(sto_all.astype(f32), (0, E_ALL_P - E_ALL), constant_values=1.0)
    lk_p = jnp.pad(log_k.astype(f32), (0, N_RXN_P - N_RXN))

    gidx_sub = pl.pallas_call(
        _prep_sub_body,
        out_shape=jax.ShapeDtypeStruct((E_SUB_P,), i32),
        grid=(E_SUB_P // EBLK,),
        in_specs=[pl.BlockSpec((EBLK,), lambda i: (i,)),
                  pl.BlockSpec((EBLK,), lambda i: (i,))],
        out_specs=pl.BlockSpec((EBLK,), lambda i: (i,)),
    )(met_sub_p, sto_sub_p)

    gidx_all = pl.pallas_call(
        _prep_all_body,
        out_shape=jax.ShapeDtypeStruct((E_ALL_P,), i32),
        grid=(E_ALL_P // EBLK,),
        in_specs=[pl.BlockSpec((EBLK,), lambda i: (i,)),
                  pl.BlockSpec((EBLK,), lambda i: (i,))],
        out_specs=pl.BlockSpec((EBLK,), lambda i: (i,)),
    )(rxn_all_p, sto_all_p)

    full = lambda shape: pl.BlockSpec(shape, lambda i: tuple(0 for _ in shape))
    T6 = pl.pallas_call(
        _stage_a_body,
        out_shape=jax.ShapeDtypeStruct((2, N_MET_P, 6), f32),
        grid=(N_MET_P // BLK,),
        in_specs=[pl.BlockSpec((BLK, 8), lambda i: (i, 0)),
                  full((2, 64)), full((64,)), full((64, 4)), full((4,))],
        out_specs=pl.BlockSpec((2, BLK, 6), lambda i: (0, i, 0)),
    )(x_p, W1, b1, W2, b2)
    T6 = T6.reshape(2 * N_MET_P, 6)

    z6 = jnp.zeros((N_RXN_P // NS, 6), f32)
    sc_b = functools.partial(
        pl.kernel,
        out_type=jax.ShapeDtypeStruct((NC, N_RXN_P, 6), f32),
        mesh=_mesh,
        scratch_types=[
            pltpu.VMEM_SHARED((2 * N_MET_P, 6), f32),
            pltpu.VMEM_SHARED((N_RXN_P, 6), f32),
            pltpu.VMEM((KB, CH), i32),
            pltpu.VMEM((KB, CH), i32),
            pltpu.VMEM((CH, 6), f32),
        ],
    )(_sc_b_body)
    A6 = sc_b(T6, gidx_sub.reshape(-1, CH), rxn_sub_p.reshape(-1, CH), z6)

    v4 = pl.pallas_call(
        _stage_c_body,
        out_shape=jax.ShapeDtypeStruct((4, N_RXN_P), f32),
        grid=(N_RXN_P // BLK,),
        in_specs=[pl.BlockSpec((2, BLK, 6), lambda i: (0, i, 0)),
                  full((4, 64)), full((64,)), full((64, 1)), full((1,)),
                  pl.BlockSpec((BLK,), lambda i: (i,))],
        out_specs=pl.BlockSpec((4, BLK), lambda i: (0, i)),
    )(A6, R1, rb1, R2, rb2, lk_p)
    v4 = v4.reshape(4 * N_RXN_P)

    z1 = jnp.zeros((N_MET_P // NS,), f32)
    sc_d = functools.partial(
        pl.kernel,
        out_type=jax.ShapeDtypeStruct((NC, N_MET_P), f32),
        mesh=_mesh,
        scratch_types=[
            pltpu.VMEM_SHARED((4 * N_RXN_P,), f32),
            pltpu.VMEM_SHARED((N_MET_P,), f32),
            pltpu.VMEM((KB, CH), i32),
            pltpu.VMEM((KB, CH), i32),
            pltpu.VMEM((CH,), f32),
        ],
    )(_sc_d_body)
    P = sc_d(v4, gidx_all.reshape(-1, CH), met_all_p.reshape(-1, CH), z1)

    dx = pl.pallas_call(
        _stage_e_body,
        out_shape=jax.ShapeDtypeStruct((N_MET_P,), f32),
        grid=(N_MET_P // BLK,),
        in_specs=[pl.BlockSpec((2, BLK), lambda i: (0, i)),
                  pl.BlockSpec((BLK,), lambda i: (i,))],
        out_specs=pl.BlockSpec((BLK,), lambda i: (i,)),
    )(P, x_p[:, 3])

    return dx[:N_MET, None]


# speedup vs baseline: 23.3846x; 23.3846x over previous
import functools

import jax
import jax.numpy as jnp
from jax import lax
from jax.experimental import pallas as pl
from jax.experimental.pallas import tpu as pltpu
from jax.experimental.pallas import tpu_sc as plsc

N_MET = 100000
N_RXN = 100000
E_SUB = 1600000
E_ALL = 3200000

N_MET_P = 100352
N_RXN_P = 100352
NC, NS = 2, 16
NW = NC * NS
CH = 128
SUB_CH_W = 392
ALL_CH_W = 784
E_SUB_P = NW * SUB_CH_W * CH
E_ALL_P = NW * ALL_CH_W * CH
BLK = 2048
EBLK = 32768
NSPARE = N_MET_P - N_MET

_mesh = plsc.VectorSubcoreMesh(core_axis_name="c", subcore_axis_name="s")


def _prep_sub_body(met_ref, sto_ref, o_ref):
    o_ref[...] = met_ref[...] + jnp.where(sto_ref[...] > 1.5, N_MET_P, 0).astype(jnp.int32)


def _prep_all_body(rxn_ref, sto_ref, o_ref):
    s = sto_ref[...]
    code = (jnp.where(jnp.abs(s) > 1.5, 2, 0) + jnp.where(s < 0.0, 1, 0)).astype(jnp.int32)
    o_ref[...] = rxn_ref[...] + code * N_RXN_P


def _stage_a_body(x_ref, W1_ref, b1_ref, W2_ref, b2_ref,
                  m0_ref, m1_ref, m2_ref, m3_ref, ext_ref):
    i = pl.program_id(0)
    xb = x_ref[...]
    conc = xb[:, 3:4]
    extc = xb[:, 4] * 2.0
    rows = i * BLK + lax.broadcasted_iota(jnp.int32, (BLK, 1), 0)
    mask = (rows < N_MET).astype(jnp.float32)
    W1 = W1_ref[...]
    msgs = []
    for s in (1.0, 2.0):
        h = jnp.tanh(conc * W1[0:1, :] + s * W1[1:2, :] + b1_ref[...][None, :])
        msgs.append((h @ W2_ref[...] + b2_ref[...][None, :]) * mask)
    for k, mref in enumerate((m0_ref, m1_ref, m2_ref, m3_ref)):
        mref[...] = jnp.stack([msgs[0][:, k], msgs[1][:, k]], axis=0)
    ext_ref[...] = extc * mask[:, 0]


def _stage_c_body(a0_ref, a1_ref, a2_ref, a3_ref, ae_ref, ac_ref,
                  R1_ref, rb1_ref, R2_ref, rb2_ref, lk_ref, o_ref):
    h = jnp.stack([a0_ref[0] + a0_ref[1], a1_ref[0] + a1_ref[1],
                   a2_ref[0] + a2_ref[1], a3_ref[0] + a3_ref[1]], axis=1)
    hh = jnp.tanh(h @ R1_ref[...] + rb1_ref[...][None, :])
    base = hh @ R2_ref[...] + rb2_ref[...][None, :]
    ext_mean = (ae_ref[0] + ae_ref[1]) / jnp.maximum(ac_ref[0] + ac_ref[1], 1.0)
    k = jnp.power(10.0, lk_ref[...])
    v = k * ext_mean * base[:, 0]
    o_ref[...] = jnp.stack([v, -v, 2.0 * v, -2.0 * v], axis=0)


def _stage_e_body(p_ref, conc_ref, o_ref):
    o_ref[...] = p_ref[0] + p_ref[1] - 0.1 * (conc_ref[...] - 1.0)


def _sc_b_body(t0_hbm, t1_hbm, t2_hbm, t3_hbm, te_hbm,
               gidx_hbm, met_hbm, rxn_hbm, z_hbm, ones_hbm,
               o0_hbm, o1_hbm, o2_hbm, o3_hbm, oe_hbm, oc_hbm,
               t0_sp, t1_sp, t2_sp, t3_sp, te_sp,
               a0_sp, a1_sp, a2_sp, a3_sp, ae_sp, ac_sp,
               gbuf, mbuf, rbuf, vals, ones_v):
    cid = lax.axis_index("c")
    sid = lax.axis_index("s")
    wid = sid * NC + cid
    t_str = 2 * N_MET_P // NS
    e_str = N_MET_P // NS
    a_str = N_RXN_P // NS
    for th, ts in ((t0_hbm, t0_sp), (t1_hbm, t1_sp), (t2_hbm, t2_sp), (t3_hbm, t3_sp)):
        pltpu.sync_copy(th.at[pl.ds(sid * t_str, t_str)], ts.at[pl.ds(sid * t_str, t_str)])
    pltpu.sync_copy(te_hbm.at[pl.ds(sid * e_str, e_str)], te_sp.at[pl.ds(sid * e_str, e_str)])
    for a_sp in (a0_sp, a1_sp, a2_sp, a3_sp, ae_sp, ac_sp):
        pltpu.sync_copy(z_hbm, a_sp.at[pl.ds(sid * a_str, a_str)])
    pltpu.sync_copy(ones_hbm, ones_v)
    plsc.subcore_barrier()

    @pl.loop(0, SUB_CH_W)
    def _(blk):
        e0 = (wid * SUB_CH_W + blk) * CH
        pltpu.sync_copy(gidx_hbm.at[pl.ds(e0, CH)], gbuf)
        pltpu.sync_copy(met_hbm.at[pl.ds(e0, CH)], mbuf)
        pltpu.sync_copy(rxn_hbm.at[pl.ds(e0, CH)], rbuf)
        for t_sp, a_sp in ((t0_sp, a0_sp), (t1_sp, a1_sp),
                           (t2_sp, a2_sp), (t3_sp, a3_sp)):
            pltpu.sync_copy(t_sp.at[gbuf], vals)
            pltpu.sync_copy(vals, a_sp.at[rbuf], add=True)
        pltpu.sync_copy(te_sp.at[mbuf], vals)
        pltpu.sync_copy(vals, ae_sp.at[rbuf], add=True)
        pltpu.sync_copy(ones_v, ac_sp.at[rbuf], add=True)

    plsc.subcore_barrier()
    for a_sp, o_hbm in ((a0_sp, o0_hbm), (a1_sp, o1_hbm), (a2_sp, o2_hbm),
                        (a3_sp, o3_hbm), (ae_sp, oe_hbm), (ac_sp, oc_hbm)):
        pltpu.sync_copy(a_sp.at[pl.ds(sid * a_str, a_str)],
                        o_hbm.at[cid, pl.ds(sid * a_str, a_str)])


def _sc_d_body(v4_hbm, gidx_hbm, met_hbm, z_hbm, out_hbm, v4_sp, dx_sp, gbuf, mbuf, vals):
    cid = lax.axis_index("c")
    sid = lax.axis_index("s")
    wid = sid * NC + cid
    v_str = 4 * N_RXN_P // NS
    d_str = N_MET_P // NS
    pltpu.sync_copy(v4_hbm.at[pl.ds(sid * v_str, v_str)],
                    v4_sp.at[pl.ds(sid * v_str, v_str)])
    pltpu.sync_copy(z_hbm, dx_sp.at[pl.ds(sid * d_str, d_str)])
    plsc.subcore_barrier()

    @pl.loop(0, ALL_CH_W)
    def _(blk):
        e0 = (wid * ALL_CH_W + blk) * CH
        pltpu.sync_copy(gidx_hbm.at[pl.ds(e0, CH)], gbuf)
        pltpu.sync_copy(met_hbm.at[pl.ds(e0, CH)], mbuf)
        pltpu.sync_copy(v4_sp.at[gbuf], vals)
        pltpu.sync_copy(vals, dx_sp.at[mbuf], add=True)

    plsc.subcore_barrier()
    pltpu.sync_copy(dx_sp.at[pl.ds(sid * d_str, d_str)],
                    out_hbm.at[cid, pl.ds(sid * d_str, d_str)])


def _pad_idx(arr, total, base):
    npad = total - arr.shape[0]
    padv = base + (jnp.arange(npad, dtype=jnp.int32) % NSPARE)
    return jnp.concatenate([arr.astype(jnp.int32), padv])


def kernel(x, met_sub, rxn_sub, sto_sub, met_all, rxn_all, sto_all,
           W1, b1, W2, b2, R1, rb1, R2, rb2, log_k):
    f32 = jnp.float32
    i32 = jnp.int32

    x_p = jnp.pad(x.astype(f32), ((0, N_MET_P - N_MET), (0, 0)))
    met_sub_p = _pad_idx(met_sub, E_SUB_P, N_MET)
    sto_sub_p = jnp.pad(sto_sub.astype(f32), (0, E_SUB_P - E_SUB), constant_values=1.0)
    rxn_sub_p = _pad_idx(rxn_sub, E_SUB_P, N_RXN)
    met_all_p = _pad_idx(met_all, E_ALL_P, N_MET)
    rxn_all_p = _pad_idx(rxn_all, E_ALL_P, N_RXN)
    sto_all_p = jnp.pad(sto_all.astype(f32), (0, E_ALL_P - E_ALL), constant_values=1.0)
    lk_p = jnp.pad(log_k.astype(f32), (0, N_RXN_P - N_RXN))

    gidx_sub = pl.pallas_call(
        _prep_sub_body,
        out_shape=jax.ShapeDtypeStruct((E_SUB_P,), i32),
        grid=(E_SUB_P // EBLK,),
        in_specs=[pl.BlockSpec((EBLK,), lambda i: (i,)),
                  pl.BlockSpec((EBLK,), lambda i: (i,))],
        out_specs=pl.BlockSpec((EBLK,), lambda i: (i,)),
    )(met_sub_p, sto_sub_p)

    gidx_all = pl.pallas_call(
        _prep_all_body,
        out_shape=jax.ShapeDtypeStruct((E_ALL_P,), i32),
        grid=(E_ALL_P // EBLK,),
        in_specs=[pl.BlockSpec((EBLK,), lambda i: (i,)),
                  pl.BlockSpec((EBLK,), lambda i: (i,))],
        out_specs=pl.BlockSpec((EBLK,), lambda i: (i,)),
    )(rxn_all_p, sto_all_p)

    full = lambda shape: pl.BlockSpec(shape, lambda i: tuple(0 for _ in shape))
    tspec = pl.BlockSpec((2, BLK), lambda i: (0, i))
    t0, t1, t2, t3, te = pl.pallas_call(
        _stage_a_body,
        out_shape=[jax.ShapeDtypeStruct((2, N_MET_P), f32)] * 4
                  + [jax.ShapeDtypeStruct((N_MET_P,), f32)],
        grid=(N_MET_P // BLK,),
        in_specs=[pl.BlockSpec((BLK, 8), lambda i: (i, 0)),
                  full((2, 64)), full((64,)), full((64, 4)), full((4,))],
        out_specs=[tspec, tspec, tspec, tspec,
                   pl.BlockSpec((BLK,), lambda i: (i,))],
    )(x_p, W1, b1, W2, b2)
    t0, t1, t2, t3 = (t.reshape(2 * N_MET_P) for t in (t0, t1, t2, t3))

    z1 = jnp.zeros((N_RXN_P // NS,), f32)
    ones_c = jnp.ones((CH,), f32)
    acc_t = jax.ShapeDtypeStruct((NC, N_RXN_P), f32)
    sc_b = functools.partial(
        pl.kernel,
        out_type=[acc_t] * 6,
        mesh=_mesh,
        scratch_types=[
            pltpu.VMEM_SHARED((2 * N_MET_P,), f32),
            pltpu.VMEM_SHARED((2 * N_MET_P,), f32),
            pltpu.VMEM_SHARED((2 * N_MET_P,), f32),
            pltpu.VMEM_SHARED((2 * N_MET_P,), f32),
            pltpu.VMEM_SHARED((N_MET_P,), f32),
            pltpu.VMEM_SHARED((N_RXN_P,), f32),
            pltpu.VMEM_SHARED((N_RXN_P,), f32),
            pltpu.VMEM_SHARED((N_RXN_P,), f32),
            pltpu.VMEM_SHARED((N_RXN_P,), f32),
            pltpu.VMEM_SHARED((N_RXN_P,), f32),
            pltpu.VMEM_SHARED((N_RXN_P,), f32),
            pltpu.VMEM((CH,), i32),
            pltpu.VMEM((CH,), i32),
            pltpu.VMEM((CH,), i32),
            pltpu.VMEM((CH,), f32),
            pltpu.VMEM((CH,), f32),
        ],
    )(_sc_b_body)
    A0, A1, A2, A3, Ae, Ac = sc_b(t0, t1, t2, t3, te,
                                  gidx_sub, met_sub_p, rxn_sub_p, z1, ones_c)

    aspec = pl.BlockSpec((2, BLK), lambda i: (0, i))
    v4 = pl.pallas_call(
        _stage_c_body,
        out_shape=jax.ShapeDtypeStruct((4, N_RXN_P), f32),
        grid=(N_RXN_P // BLK,),
        in_specs=[aspec, aspec, aspec, aspec, aspec, aspec,
                  full((4, 64)), full((64,)), full((64, 1)), full((1,)),
                  pl.BlockSpec((BLK,), lambda i: (i,))],
        out_specs=pl.BlockSpec((4, BLK), lambda i: (0, i)),
    )(A0, A1, A2, A3, Ae, Ac, R1, rb1, R2, rb2, lk_p)
    v4 = v4.reshape(4 * N_RXN_P)

    zd = jnp.zeros((N_MET_P // NS,), f32)
    sc_d = functools.partial(
        pl.kernel,
        out_type=jax.ShapeDtypeStruct((NC, N_MET_P), f32),
        mesh=_mesh,
        scratch_types=[
            pltpu.VMEM_SHARED((4 * N_RXN_P,), f32),
            pltpu.VMEM_SHARED((N_MET_P,), f32),
            pltpu.VMEM((CH,), i32),
            pltpu.VMEM((CH,), i32),
            pltpu.VMEM((CH,), f32),
        ],
    )(_sc_d_body)
    P = sc_d(v4, gidx_all, met_all_p, zd)

    dx = pl.pallas_call(
        _stage_e_body,
        out_shape=jax.ShapeDtypeStruct((N_MET_P,), f32),
        grid=(N_MET_P // BLK,),
        in_specs=[pl.BlockSpec((2, BLK), lambda i: (0, i)),
                  pl.BlockSpec((BLK,), lambda i: (i,))],
        out_specs=pl.BlockSpec((BLK,), lambda i: (i,)),
    )(P, x_p[:, 3])

    return dx[:N_MET, None]


# speedup vs baseline: 57.2789x; 2.4494x over previous
import functools

import jax
import jax.numpy as jnp
from jax import lax
from jax.experimental import pallas as pl
from jax.experimental.pallas import tpu as pltpu
from jax.experimental.pallas import tpu_sc as plsc

N_MET = 100000
N_RXN = 100000
E_SUB = 1600000
E_ALL = 3200000

N_MET_P = 100352
N_RXN_P = 100352
NC, NS = 2, 16
NW = NC * NS
CH = 128
SUB_CH_W = 392
ALL_CH_W = 784
E_SUB_P = NW * SUB_CH_W * CH
E_ALL_P = NW * ALL_CH_W * CH
BLK = 2048
EBLK = 32768
NSPARE = N_MET_P - N_MET

_mesh = plsc.VectorSubcoreMesh(core_axis_name="c", subcore_axis_name="s")


def _prep_sub_body(met_ref, sto_ref, o_ref):
    o_ref[...] = met_ref[...] + jnp.where(sto_ref[...] > 1.5, N_MET_P, 0).astype(jnp.int32)


def _prep_all_body(rxn_ref, sto_ref, o_ref):
    s = sto_ref[...]
    code = (jnp.where(jnp.abs(s) > 1.5, 2, 0) + jnp.where(s < 0.0, 1, 0)).astype(jnp.int32)
    o_ref[...] = rxn_ref[...] + code * N_RXN_P


def _stage_a_body(x_ref, W1_ref, b1_ref, W2_ref, b2_ref,
                  m0_ref, m1_ref, m2_ref, m3_ref, ext_ref):
    i = pl.program_id(0)
    xb = x_ref[...]
    conc = xb[:, 3:4]
    extc = xb[:, 4] * 2.0
    rows = i * BLK + lax.broadcasted_iota(jnp.int32, (BLK, 1), 0)
    mask = (rows < N_MET).astype(jnp.float32)
    W1 = W1_ref[...]
    msgs = []
    for s in (1.0, 2.0):
        h = jnp.tanh(conc * W1[0:1, :] + s * W1[1:2, :] + b1_ref[...][None, :])
        msgs.append((h @ W2_ref[...] + b2_ref[...][None, :]) * mask)
    for k, mref in enumerate((m0_ref, m1_ref, m2_ref, m3_ref)):
        mref[...] = jnp.stack([msgs[0][:, k], msgs[1][:, k]], axis=0)
    ext_ref[...] = extc * mask[:, 0]


def _stage_c_body(a0_ref, a1_ref, a2_ref, a3_ref, ae_ref, ac_ref,
                  R1_ref, rb1_ref, R2_ref, rb2_ref, lk_ref, o_ref):
    h = jnp.stack([a0_ref[0] + a0_ref[1], a1_ref[0] + a1_ref[1],
                   a2_ref[0] + a2_ref[1], a3_ref[0] + a3_ref[1]], axis=1)
    hh = jnp.tanh(h @ R1_ref[...] + rb1_ref[...][None, :])
    base = hh @ R2_ref[...] + rb2_ref[...][None, :]
    ext_mean = (ae_ref[0] + ae_ref[1]) / jnp.maximum(ac_ref[0] + ac_ref[1], 1.0)
    k = jnp.power(10.0, lk_ref[...])
    v = k * ext_mean * base[:, 0]
    o_ref[...] = jnp.stack([v, -v, 2.0 * v, -2.0 * v], axis=0)


def _stage_e_body(p_ref, conc_ref, o_ref):
    o_ref[...] = p_ref[0] + p_ref[1] - 0.1 * (conc_ref[...] - 1.0)


def _sc_b_body(t0_hbm, t1_hbm, t2_hbm, t3_hbm, te_hbm,
               gidx_hbm, met_hbm, rxn_hbm, z_hbm, ones_hbm,
               o0_hbm, o1_hbm, o2_hbm, o3_hbm, oe_hbm, oc_hbm,
               t0_sp, t1_sp, t2_sp, t3_sp, te_sp,
               a0_sp, a1_sp, a2_sp, a3_sp, ae_sp, ac_sp,
               g0, g1, m0, m1, r0, r1,
               v00, v01, v10, v11, v20, v21, v30, v31, ve0, ve1,
               ones_v, si0, si1, sg0, sg1, ss0, ss1):
    gbuf, mbuf, rbuf = (g0, g1), (m0, m1), (r0, r1)
    v0, v1, v2, v3, ve = (v00, v01), (v10, v11), (v20, v21), (v30, v31), (ve0, ve1)
    si, sg, ss = (si0, si1), (sg0, sg1), (ss0, ss1)
    cid = lax.axis_index("c")
    sid = lax.axis_index("s")
    wid = sid * NC + cid
    t_str = 2 * N_MET_P // NS
    e_str = N_MET_P // NS
    a_str = N_RXN_P // NS
    for th, ts in ((t0_hbm, t0_sp), (t1_hbm, t1_sp), (t2_hbm, t2_sp), (t3_hbm, t3_sp)):
        pltpu.sync_copy(th.at[pl.ds(sid * t_str, t_str)], ts.at[pl.ds(sid * t_str, t_str)])
    pltpu.sync_copy(te_hbm.at[pl.ds(sid * e_str, e_str)], te_sp.at[pl.ds(sid * e_str, e_str)])
    for a_sp in (a0_sp, a1_sp, a2_sp, a3_sp, ae_sp, ac_sp):
        pltpu.sync_copy(z_hbm, a_sp.at[pl.ds(sid * a_str, a_str)])
    pltpu.sync_copy(ones_hbm, ones_v)
    plsc.subcore_barrier()

    @pl.loop(0, SUB_CH_W // 2)
    def _(blk):
        hi = []
        for s in range(2):
            e0 = (wid * SUB_CH_W + blk * 2 + s) * CH
            hi.append((pltpu.async_copy(gidx_hbm.at[pl.ds(e0, CH)], gbuf[s], si[s]),
                       pltpu.async_copy(met_hbm.at[pl.ds(e0, CH)], mbuf[s], si[s]),
                       pltpu.async_copy(rxn_hbm.at[pl.ds(e0, CH)], rbuf[s], si[s])))
        hg = []
        for s in range(2):
            for h in hi[s]:
                h.wait()
            hg.append([pltpu.async_copy(t_sp.at[gbuf[s]], v_b, sg[s])
                       for t_sp, v_b in ((t0_sp, v0[s]), (t1_sp, v1[s]),
                                         (t2_sp, v2[s]), (t3_sp, v3[s]))]
                      + [pltpu.async_copy(te_sp.at[mbuf[s]], ve[s], sg[s])])
        hs = []
        for s in range(2):
            for h in hg[s]:
                h.wait()
            hs.append([pltpu.async_copy(v_b, a_sp.at[rbuf[s]], ss[s], add=True)
                       for v_b, a_sp in ((v0[s], a0_sp), (v1[s], a1_sp),
                                         (v2[s], a2_sp), (v3[s], a3_sp),
                                         (ve[s], ae_sp))]
                      + [pltpu.async_copy(ones_v, ac_sp.at[rbuf[s]], ss[s], add=True)])
        for s in range(2):
            for h in hs[s]:
                h.wait()

    plsc.subcore_barrier()
    for a_sp, o_hbm in ((a0_sp, o0_hbm), (a1_sp, o1_hbm), (a2_sp, o2_hbm),
                        (a3_sp, o3_hbm), (ae_sp, oe_hbm), (ac_sp, oc_hbm)):
        pltpu.sync_copy(a_sp.at[pl.ds(sid * a_str, a_str)],
                        o_hbm.at[cid, pl.ds(sid * a_str, a_str)])


def _sc_d_body(v4_hbm, gidx_hbm, met_hbm, z_hbm, out_hbm, v4_sp, dx_sp,
               g0, g1, g2, g3, m0, m1, m2, m3, w0, w1, w2, w3,
               si0, si1, si2, si3, sg0, sg1, sg2, sg3, ss0, ss1, ss2, ss3):
    gbuf, mbuf, vals = (g0, g1, g2, g3), (m0, m1, m2, m3), (w0, w1, w2, w3)
    si, sg, ss = (si0, si1, si2, si3), (sg0, sg1, sg2, sg3), (ss0, ss1, ss2, ss3)
    cid = lax.axis_index("c")
    sid = lax.axis_index("s")
    wid = sid * NC + cid
    v_str = 4 * N_RXN_P // NS
    d_str = N_MET_P // NS
    pltpu.sync_copy(v4_hbm.at[pl.ds(sid * v_str, v_str)],
                    v4_sp.at[pl.ds(sid * v_str, v_str)])
    pltpu.sync_copy(z_hbm, dx_sp.at[pl.ds(sid * d_str, d_str)])
    plsc.subcore_barrier()

    @pl.loop(0, ALL_CH_W // 4)
    def _(blk):
        hi = []
        for s in range(4):
            e0 = (wid * ALL_CH_W + blk * 4 + s) * CH
            hi.append((pltpu.async_copy(gidx_hbm.at[pl.ds(e0, CH)], gbuf[s], si[s]),
                       pltpu.async_copy(met_hbm.at[pl.ds(e0, CH)], mbuf[s], si[s])))
        hg = []
        for s in range(4):
            hi[s][0].wait()
            hi[s][1].wait()
            hg.append(pltpu.async_copy(v4_sp.at[gbuf[s]], vals[s], sg[s]))
        hs = []
        for s in range(4):
            hg[s].wait()
            hs.append(pltpu.async_copy(vals[s], dx_sp.at[mbuf[s]], ss[s], add=True))
        for h in hs:
            h.wait()

    plsc.subcore_barrier()
    pltpu.sync_copy(dx_sp.at[pl.ds(sid * d_str, d_str)],
                    out_hbm.at[cid, pl.ds(sid * d_str, d_str)])


def _pad_idx(arr, total, base):
    npad = total - arr.shape[0]
    padv = base + (jnp.arange(npad, dtype=jnp.int32) % NSPARE)
    return jnp.concatenate([arr.astype(jnp.int32), padv])


def kernel(x, met_sub, rxn_sub, sto_sub, met_all, rxn_all, sto_all,
           W1, b1, W2, b2, R1, rb1, R2, rb2, log_k):
    f32 = jnp.float32
    i32 = jnp.int32

    x_p = jnp.pad(x.astype(f32), ((0, N_MET_P - N_MET), (0, 0)))
    met_sub_p = _pad_idx(met_sub, E_SUB_P, N_MET)
    sto_sub_p = jnp.pad(sto_sub.astype(f32), (0, E_SUB_P - E_SUB), constant_values=1.0)
    rxn_sub_p = _pad_idx(rxn_sub, E_SUB_P, N_RXN)
    met_all_p = _pad_idx(met_all, E_ALL_P, N_MET)
    rxn_all_p = _pad_idx(rxn_all, E_ALL_P, N_RXN)
    sto_all_p = jnp.pad(sto_all.astype(f32), (0, E_ALL_P - E_ALL), constant_values=1.0)
    lk_p = jnp.pad(log_k.astype(f32), (0, N_RXN_P - N_RXN))

    gidx_sub = pl.pallas_call(
        _prep_sub_body,
        out_shape=jax.ShapeDtypeStruct((E_SUB_P,), i32),
        grid=(E_SUB_P // EBLK,),
        in_specs=[pl.BlockSpec((EBLK,), lambda i: (i,)),
                  pl.BlockSpec((EBLK,), lambda i: (i,))],
        out_specs=pl.BlockSpec((EBLK,), lambda i: (i,)),
    )(met_sub_p, sto_sub_p)

    gidx_all = pl.pallas_call(
        _prep_all_body,
        out_shape=jax.ShapeDtypeStruct((E_ALL_P,), i32),
        grid=(E_ALL_P // EBLK,),
        in_specs=[pl.BlockSpec((EBLK,), lambda i: (i,)),
                  pl.BlockSpec((EBLK,), lambda i: (i,))],
        out_specs=pl.BlockSpec((EBLK,), lambda i: (i,)),
    )(rxn_all_p, sto_all_p)

    full = lambda shape: pl.BlockSpec(shape, lambda i: tuple(0 for _ in shape))
    tspec = pl.BlockSpec((2, BLK), lambda i: (0, i))
    t0, t1, t2, t3, te = pl.pallas_call(
        _stage_a_body,
        out_shape=[jax.ShapeDtypeStruct((2, N_MET_P), f32)] * 4
                  + [jax.ShapeDtypeStruct((N_MET_P,), f32)],
        grid=(N_MET_P // BLK,),
        in_specs=[pl.BlockSpec((BLK, 8), lambda i: (i, 0)),
                  full((2, 64)), full((64,)), full((64, 4)), full((4,))],
        out_specs=[tspec, tspec, tspec, tspec,
                   pl.BlockSpec((BLK,), lambda i: (i,))],
    )(x_p, W1, b1, W2, b2)
    t0, t1, t2, t3 = (t.reshape(2 * N_MET_P) for t in (t0, t1, t2, t3))

    z1 = jnp.zeros((N_RXN_P // NS,), f32)
    ones_c = jnp.ones((CH,), f32)
    acc_t = jax.ShapeDtypeStruct((NC, N_RXN_P), f32)
    sc_b = functools.partial(
        pl.kernel,
        out_type=[acc_t] * 6,
        mesh=_mesh,
        scratch_types=[
            pltpu.VMEM_SHARED((2 * N_MET_P,), f32),
            pltpu.VMEM_SHARED((2 * N_MET_P,), f32),
            pltpu.VMEM_SHARED((2 * N_MET_P,), f32),
            pltpu.VMEM_SHARED((2 * N_MET_P,), f32),
            pltpu.VMEM_SHARED((N_MET_P,), f32),
            pltpu.VMEM_SHARED((N_RXN_P,), f32),
            pltpu.VMEM_SHARED((N_RXN_P,), f32),
            pltpu.VMEM_SHARED((N_RXN_P,), f32),
            pltpu.VMEM_SHARED((N_RXN_P,), f32),
            pltpu.VMEM_SHARED((N_RXN_P,), f32),
            pltpu.VMEM_SHARED((N_RXN_P,), f32),
        ] + [pltpu.VMEM((CH,), i32)] * 6
          + [pltpu.VMEM((CH,), f32)] * 10
          + [pltpu.VMEM((CH,), f32)]
          + [pltpu.SemaphoreType.DMA] * 6,
    )(_sc_b_body)
    A0, A1, A2, A3, Ae, Ac = sc_b(t0, t1, t2, t3, te,
                                  gidx_sub, met_sub_p, rxn_sub_p, z1, ones_c)

    aspec = pl.BlockSpec((2, BLK), lambda i: (0, i))
    v4 = pl.pallas_call(
        _stage_c_body,
        out_shape=jax.ShapeDtypeStruct((4, N_RXN_P), f32),
        grid=(N_RXN_P // BLK,),
        in_specs=[aspec, aspec, aspec, aspec, aspec, aspec,
                  full((4, 64)), full((64,)), full((64, 1)), full((1,)),
                  pl.BlockSpec((BLK,), lambda i: (i,))],
        out_specs=pl.BlockSpec((4, BLK), lambda i: (0, i)),
    )(A0, A1, A2, A3, Ae, Ac, R1, rb1, R2, rb2, lk_p)
    v4 = v4.reshape(4 * N_RXN_P)

    zd = jnp.zeros((N_MET_P // NS,), f32)
    sc_d = functools.partial(
        pl.kernel,
        out_type=jax.ShapeDtypeStruct((NC, N_MET_P), f32),
        mesh=_mesh,
        scratch_types=[
            pltpu.VMEM_SHARED((4 * N_RXN_P,), f32),
            pltpu.VMEM_SHARED((N_MET_P,), f32),
        ] + [pltpu.VMEM((CH,), i32)] * 8
          + [pltpu.VMEM((CH,), f32)] * 4
          + [pltpu.SemaphoreType.DMA] * 12,
    )(_sc_d_body)
    P = sc_d(v4, gidx_all, met_all_p, zd)

    dx = pl.pallas_call(
        _stage_e_body,
        out_shape=jax.ShapeDtypeStruct((N_MET_P,), f32),
        grid=(N_MET_P // BLK,),
        in_specs=[pl.BlockSpec((2, BLK), lambda i: (0, i)),
                  pl.BlockSpec((BLK,), lambda i: (i,))],
        out_specs=pl.BlockSpec((BLK,), lambda i: (i,)),
    )(P, x_p[:, 3])

    return dx[:N_MET, None]


# speedup vs baseline: 61.3502x; 1.0711x over previous
import functools

import jax
import jax.numpy as jnp
from jax import lax
from jax.experimental import pallas as pl
from jax.experimental.pallas import tpu as pltpu
from jax.experimental.pallas import tpu_sc as plsc

N_MET = 100000
N_RXN = 100000
E_SUB = 1600000
E_ALL = 3200000

N_MET_P = 100352
N_RXN_P = 100352
NC, NS = 2, 16
NW = NC * NS
CH = 128
SUB_CH_W = 392
ALL_CH_W = 784
E_SUB_P = NW * SUB_CH_W * CH
E_ALL_P = NW * ALL_CH_W * CH
BLK = 2048
EBLK = 32768
E_SUB_P2 = 50 * EBLK
E_ALL_P2 = 99 * EBLK
NSPARE = N_MET_P - N_MET

_mesh = plsc.VectorSubcoreMesh(core_axis_name="c", subcore_axis_name="s")


def _prep_sub_body(met_ref, sto_ref, rxn_ref, gidx_ref, met_o_ref, rxn_o_ref):
    i = pl.program_id(0)
    pos = i * EBLK + lax.broadcasted_iota(jnp.int32, (EBLK,), 0)
    valid = pos < E_SUB
    spare = pos % NSPARE
    gidx = met_ref[...] + jnp.where(sto_ref[...] > 1.5, N_MET_P, 0).astype(jnp.int32)
    gidx_ref[...] = jnp.where(valid, gidx, N_MET + spare)
    met_o_ref[...] = jnp.where(valid, met_ref[...], N_MET + spare)
    rxn_o_ref[...] = jnp.where(valid, rxn_ref[...], N_RXN + spare)


def _prep_all_body(rxn_ref, sto_ref, met_ref, gidx_ref, met_o_ref):
    i = pl.program_id(0)
    pos = i * EBLK + lax.broadcasted_iota(jnp.int32, (EBLK,), 0)
    valid = pos < E_ALL
    spare = pos % NSPARE
    s = sto_ref[...]
    code = (jnp.where(jnp.abs(s) > 1.5, 2, 0) + jnp.where(s < 0.0, 1, 0)).astype(jnp.int32)
    gidx_ref[...] = jnp.where(valid, rxn_ref[...] + code * N_RXN_P, N_RXN + spare)
    met_o_ref[...] = jnp.where(valid, met_ref[...], N_MET + spare)


def _stage_a_body(conc_ref, exti_ref, W1_ref, b1_ref, W2_ref, b2_ref,
                  m0_ref, m1_ref, m2_ref, m3_ref, ext_ref):
    i = pl.program_id(0)
    conc = conc_ref[...][:, None]
    rows = i * BLK + lax.broadcasted_iota(jnp.int32, (BLK, 1), 0)
    mask = (rows < N_MET).astype(jnp.float32)
    W1 = W1_ref[...]
    msgs = []
    for s in (1.0, 2.0):
        h = jnp.tanh(conc * W1[0:1, :] + s * W1[1:2, :] + b1_ref[...][None, :])
        msgs.append((h @ W2_ref[...] + b2_ref[...][None, :]) * mask)
    for k, mref in enumerate((m0_ref, m1_ref, m2_ref, m3_ref)):
        mref[...] = jnp.stack([msgs[0][:, k], msgs[1][:, k]], axis=0)
    ext_ref[...] = exti_ref[...] * 2.0 * mask[:, 0]


def _stage_c_body(a0_ref, a1_ref, a2_ref, a3_ref, ae_ref, ac_ref,
                  R1_ref, rb1_ref, R2_ref, rb2_ref, lk_ref, o_ref):
    h = jnp.stack([a0_ref[0] + a0_ref[1], a1_ref[0] + a1_ref[1],
                   a2_ref[0] + a2_ref[1], a3_ref[0] + a3_ref[1]], axis=1)
    hh = jnp.tanh(h @ R1_ref[...] + rb1_ref[...][None, :])
    base = hh @ R2_ref[...] + rb2_ref[...][None, :]
    ext_mean = (ae_ref[0] + ae_ref[1]) / jnp.maximum(ac_ref[0] + ac_ref[1], 1.0)
    k = jnp.power(10.0, lk_ref[...])
    v = k * ext_mean * base[:, 0]
    o_ref[...] = jnp.stack([v, -v, 2.0 * v, -2.0 * v], axis=0)


def _stage_e_body(p_ref, conc_ref, o_ref):
    o_ref[...] = p_ref[0] + p_ref[1] - 0.1 * (conc_ref[...] - 1.0)


def _sc_b_body(t0_hbm, t1_hbm, t2_hbm, t3_hbm, te_hbm,
               gidx_hbm, met_hbm, rxn_hbm, z_hbm, ones_hbm,
               o0_hbm, o1_hbm, o2_hbm, o3_hbm, oe_hbm, oc_hbm,
               t0_sp, t1_sp, t2_sp, t3_sp, te_sp,
               a0_sp, a1_sp, a2_sp, a3_sp, ae_sp, ac_sp,
               g0, g1, m0, m1, r0, r1,
               v00, v01, v10, v11, v20, v21, v30, v31, ve0, ve1,
               ones_v, si0, si1, sg0, sg1, ss0, ss1):
    gbuf, mbuf, rbuf = (g0, g1), (m0, m1), (r0, r1)
    v0, v1, v2, v3, ve = (v00, v01), (v10, v11), (v20, v21), (v30, v31), (ve0, ve1)
    si, sg, ss = (si0, si1), (sg0, sg1), (ss0, ss1)
    cid = lax.axis_index("c")
    sid = lax.axis_index("s")
    wid = sid * NC + cid
    t_str = 2 * N_MET_P // NS
    e_str = N_MET_P // NS
    a_str = N_RXN_P // NS
    for th, ts in ((t0_hbm, t0_sp), (t1_hbm, t1_sp), (t2_hbm, t2_sp), (t3_hbm, t3_sp)):
        pltpu.sync_copy(th.at[pl.ds(sid * t_str, t_str)], ts.at[pl.ds(sid * t_str, t_str)])
    pltpu.sync_copy(te_hbm.at[pl.ds(sid * e_str, e_str)], te_sp.at[pl.ds(sid * e_str, e_str)])
    for a_sp in (a0_sp, a1_sp, a2_sp, a3_sp, ae_sp, ac_sp):
        pltpu.sync_copy(z_hbm, a_sp.at[pl.ds(sid * a_str, a_str)])
    pltpu.sync_copy(ones_hbm, ones_v)
    plsc.subcore_barrier()

    def _fire_idx(chunk, s):
        e0 = chunk * CH
        return (pltpu.async_copy(gidx_hbm.at[pl.ds(e0, CH)], gbuf[s], si[s]),
                pltpu.async_copy(met_hbm.at[pl.ds(e0, CH)], mbuf[s], si[s]),
                pltpu.async_copy(rxn_hbm.at[pl.ds(e0, CH)], rbuf[s], si[s]))

    base = wid * SUB_CH_W
    for s in range(2):
        _fire_idx(base + s, s)

    @pl.loop(0, SUB_CH_W // 2)
    def _(blk):
        hg = []
        for s in range(2):
            for trg in (gbuf[s], mbuf[s], rbuf[s]):
                pltpu.make_async_copy(gidx_hbm.at[pl.ds(0, CH)], trg, si[s]).wait()
            hg.append([pltpu.async_copy(t_sp.at[gbuf[s]], v_b, sg[s])
                       for t_sp, v_b in ((t0_sp, v0[s]), (t1_sp, v1[s]),
                                         (t2_sp, v2[s]), (t3_sp, v3[s]))]
                      + [pltpu.async_copy(te_sp.at[mbuf[s]], ve[s], sg[s])])
        hs = []
        for s in range(2):
            for h in hg[s]:
                h.wait()
            hs.append([pltpu.async_copy(v_b, a_sp.at[rbuf[s]], ss[s], add=True)
                       for v_b, a_sp in ((v0[s], a0_sp), (v1[s], a1_sp),
                                         (v2[s], a2_sp), (v3[s], a3_sp),
                                         (ve[s], ae_sp))]
                      + [pltpu.async_copy(ones_v, ac_sp.at[rbuf[s]], ss[s], add=True)])
        for s in range(2):
            for h in hs[s]:
                h.wait()
            _fire_idx(base + (blk + 1) * 2 + s, s)

    for s in range(2):
        for trg in (gbuf[s], mbuf[s], rbuf[s]):
            pltpu.make_async_copy(gidx_hbm.at[pl.ds(0, CH)], trg, si[s]).wait()
    plsc.subcore_barrier()
    for a_sp, o_hbm in ((a0_sp, o0_hbm), (a1_sp, o1_hbm), (a2_sp, o2_hbm),
                        (a3_sp, o3_hbm), (ae_sp, oe_hbm), (ac_sp, oc_hbm)):
        pltpu.sync_copy(a_sp.at[pl.ds(sid * a_str, a_str)],
                        o_hbm.at[cid, pl.ds(sid * a_str, a_str)])


def _sc_d_body(v4_hbm, gidx_hbm, met_hbm, z_hbm, out_hbm, v4_sp, dx_sp,
               g0, g1, g2, g3, m0, m1, m2, m3, w0, w1, w2, w3,
               si0, si1, si2, si3, sg0, sg1, sg2, sg3, ss0, ss1, ss2, ss3):
    gbuf, mbuf, vals = (g0, g1, g2, g3), (m0, m1, m2, m3), (w0, w1, w2, w3)
    si, sg, ss = (si0, si1, si2, si3), (sg0, sg1, sg2, sg3), (ss0, ss1, ss2, ss3)
    cid = lax.axis_index("c")
    sid = lax.axis_index("s")
    wid = sid * NC + cid
    v_str = 4 * N_RXN_P // NS
    d_str = N_MET_P // NS
    pltpu.sync_copy(v4_hbm.at[pl.ds(sid * v_str, v_str)],
                    v4_sp.at[pl.ds(sid * v_str, v_str)])
    pltpu.sync_copy(z_hbm, dx_sp.at[pl.ds(sid * d_str, d_str)])
    plsc.subcore_barrier()

    def _fire_idx(chunk, s):
        e0 = chunk * CH
        return (pltpu.async_copy(gidx_hbm.at[pl.ds(e0, CH)], gbuf[s], si[s]),
                pltpu.async_copy(met_hbm.at[pl.ds(e0, CH)], mbuf[s], si[s]))

    base = wid * ALL_CH_W
    for s in range(4):
        _fire_idx(base + s, s)

    @pl.loop(0, ALL_CH_W // 4)
    def _(blk):
        hg = []
        for s in range(4):
            for trg in (gbuf[s], mbuf[s]):
                pltpu.make_async_copy(gidx_hbm.at[pl.ds(0, CH)], trg, si[s]).wait()
            hg.append(pltpu.async_copy(v4_sp.at[gbuf[s]], vals[s], sg[s]))
        hs = []
        for s in range(4):
            hg[s].wait()
            hs.append(pltpu.async_copy(vals[s], dx_sp.at[mbuf[s]], ss[s], add=True))
        for s in range(4):
            hs[s].wait()
            _fire_idx(base + (blk + 1) * 4 + s, s)

    for s in range(4):
        for trg in (gbuf[s], mbuf[s]):
            pltpu.make_async_copy(gidx_hbm.at[pl.ds(0, CH)], trg, si[s]).wait()
    plsc.subcore_barrier()
    pltpu.sync_copy(dx_sp.at[pl.ds(sid * d_str, d_str)],
                    out_hbm.at[cid, pl.ds(sid * d_str, d_str)])


def kernel(x, met_sub, rxn_sub, sto_sub, met_all, rxn_all, sto_all,
           W1, b1, W2, b2, R1, rb1, R2, rb2, log_k):
    f32 = jnp.float32
    i32 = jnp.int32

    conc_p = jnp.pad(x[:, 3].astype(f32), (0, N_MET_P - N_MET))
    exti_p = jnp.pad(x[:, 4].astype(f32), (0, N_MET_P - N_MET))
    lk_p = jnp.pad(log_k.astype(f32), (0, N_RXN_P - N_RXN))

    nsb = E_SUB // EBLK
    sub_in = pl.BlockSpec((EBLK,), lambda i: (jnp.minimum(i, nsb),))
    out_sp = pl.BlockSpec((EBLK,), lambda i: (i,))
    gidx_sub, met_sub_o, rxn_sub_o = pl.pallas_call(
        _prep_sub_body,
        out_shape=[jax.ShapeDtypeStruct((E_SUB_P2,), i32)] * 3,
        grid=(E_SUB_P2 // EBLK,),
        in_specs=[sub_in, sub_in, sub_in],
        out_specs=[out_sp, out_sp, out_sp],
    )(met_sub.astype(i32), sto_sub.astype(f32), rxn_sub.astype(i32))

    nab = E_ALL // EBLK
    all_in = pl.BlockSpec((EBLK,), lambda i: (jnp.minimum(i, nab),))
    gidx_all, met_all_o = pl.pallas_call(
        _prep_all_body,
        out_shape=[jax.ShapeDtypeStruct((E_ALL_P2,), i32)] * 2,
        grid=(E_ALL_P2 // EBLK,),
        in_specs=[all_in, all_in, all_in],
        out_specs=[out_sp, out_sp],
    )(rxn_all.astype(i32), sto_all.astype(f32), met_all.astype(i32))

    full = lambda shape: pl.BlockSpec(shape, lambda i: tuple(0 for _ in shape))
    tspec = pl.BlockSpec((2, BLK), lambda i: (0, i))
    t0, t1, t2, t3, te = pl.pallas_call(
        _stage_a_body,
        out_shape=[jax.ShapeDtypeStruct((2, N_MET_P), f32)] * 4
                  + [jax.ShapeDtypeStruct((N_MET_P,), f32)],
        grid=(N_MET_P // BLK,),
        in_specs=[pl.BlockSpec((BLK,), lambda i: (i,)),
                  pl.BlockSpec((BLK,), lambda i: (i,)),
                  full((2, 64)), full((64,)), full((64, 4)), full((4,))],
        out_specs=[tspec, tspec, tspec, tspec,
                   pl.BlockSpec((BLK,), lambda i: (i,))],
    )(conc_p, exti_p, W1, b1, W2, b2)
    t0, t1, t2, t3 = (t.reshape(2 * N_MET_P) for t in (t0, t1, t2, t3))

    z1 = jnp.zeros((N_RXN_P // NS,), f32)
    ones_c = jnp.ones((CH,), f32)
    acc_t = jax.ShapeDtypeStruct((NC, N_RXN_P), f32)
    sc_b = functools.partial(
        pl.kernel,
        out_type=[acc_t] * 6,
        mesh=_mesh,
        scratch_types=[
            pltpu.VMEM_SHARED((2 * N_MET_P,), f32),
            pltpu.VMEM_SHARED((2 * N_MET_P,), f32),
            pltpu.VMEM_SHARED((2 * N_MET_P,), f32),
            pltpu.VMEM_SHARED((2 * N_MET_P,), f32),
            pltpu.VMEM_SHARED((N_MET_P,), f32),
            pltpu.VMEM_SHARED((N_RXN_P,), f32),
            pltpu.VMEM_SHARED((N_RXN_P,), f32),
            pltpu.VMEM_SHARED((N_RXN_P,), f32),
            pltpu.VMEM_SHARED((N_RXN_P,), f32),
            pltpu.VMEM_SHARED((N_RXN_P,), f32),
            pltpu.VMEM_SHARED((N_RXN_P,), f32),
        ] + [pltpu.VMEM((CH,), i32)] * 6
          + [pltpu.VMEM((CH,), f32)] * 10
          + [pltpu.VMEM((CH,), f32)]
          + [pltpu.SemaphoreType.DMA] * 6,
    )(_sc_b_body)
    A0, A1, A2, A3, Ae, Ac = sc_b(t0, t1, t2, t3, te,
                                  gidx_sub, met_sub_o, rxn_sub_o, z1, ones_c)

    aspec = pl.BlockSpec((2, BLK), lambda i: (0, i))
    v4 = pl.pallas_call(
        _stage_c_body,
        out_shape=jax.ShapeDtypeStruct((4, N_RXN_P), f32),
        grid=(N_RXN_P // BLK,),
        in_specs=[aspec, aspec, aspec, aspec, aspec, aspec,
                  full((4, 64)), full((64,)), full((64, 1)), full((1,)),
                  pl.BlockSpec((BLK,), lambda i: (i,))],
        out_specs=pl.BlockSpec((4, BLK), lambda i: (0, i)),
    )(A0, A1, A2, A3, Ae, Ac, R1, rb1, R2, rb2, lk_p)
    v4 = v4.reshape(4 * N_RXN_P)

    zd = jnp.zeros((N_MET_P // NS,), f32)
    sc_d = functools.partial(
        pl.kernel,
        out_type=jax.ShapeDtypeStruct((NC, N_MET_P), f32),
        mesh=_mesh,
        scratch_types=[
            pltpu.VMEM_SHARED((4 * N_RXN_P,), f32),
            pltpu.VMEM_SHARED((N_MET_P,), f32),
        ] + [pltpu.VMEM((CH,), i32)] * 8
          + [pltpu.VMEM((CH,), f32)] * 4
          + [pltpu.SemaphoreType.DMA] * 12,
    )(_sc_d_body)
    P = sc_d(v4, gidx_all, met_all_o, zd)

    dx = pl.pallas_call(
        _stage_e_body,
        out_shape=jax.ShapeDtypeStruct((N_MET_P,), f32),
        grid=(N_MET_P // BLK,),
        in_specs=[pl.BlockSpec((2, BLK), lambda i: (0, i)),
                  pl.BlockSpec((BLK,), lambda i: (i,))],
        out_specs=pl.BlockSpec((BLK,), lambda i: (i,)),
    )(P, conc_p)

    return dx[:N_MET, None]
